# baseline (speedup 1.0000x reference)
import jax
import jax.numpy as jnp
from jax.experimental import pallas as pl
from jax.experimental.pallas import tpu as pltpu
from jax.experimental.pallas import tpu_sc as plsc

T, D, E, K = 2048, 1024, 8, 2
F, FS = 1024, 4096
A = T * K
M = 256
NT = 23
NP = NT * M
EPS = 1e-6
SCW = 128
HD = D // 2
LANES = 128


def _rms(x):
    var = jnp.mean(x * x, axis=-1, keepdims=True)
    return x * jax.lax.rsqrt(var + EPS)


def _router_kernel(orig_ref, pre2_ref, pref_ref, rw_ref,
                   rin_ref, pos_ref, topw_ref, meta_ref, lb_ref):
    x = orig_ref[...]
    xn = _rms(x)
    rin_ref[...] = (xn * pre2_ref[...]).astype(jnp.bfloat16)
    gate = xn * (D ** -0.5) * pref_ref[...]
    logits = jnp.dot(gate, rw_ref[...], preferred_element_type=jnp.float32)
    li = jax.lax.broadcasted_iota(jnp.int32, (T, LANES), 1)
    lmask = li < E
    lm = jnp.where(lmask, logits, -1e30)
    mx = jnp.max(lm, axis=1, keepdims=True)
    ex = jnp.where(lmask, jnp.exp(lm - mx), 0.0)
    probs = ex / jnp.sum(ex, axis=1, keepdims=True)
    p0 = jnp.max(probs, axis=1, keepdims=True)
    i0 = jnp.min(jnp.where(probs == p0, li, LANES), axis=1, keepdims=True)
    pmask = jnp.where(li == i0, -1.0, probs)
    p1 = jnp.max(pmask, axis=1, keepdims=True)
    i1 = jnp.min(jnp.where(pmask == p1, li, LANES), axis=1, keepdims=True)
    s = p0 + p1
    topw_ref[...] = jnp.concatenate([p0 / s, p1 / s], axis=1)
    oh0 = (li == i0).astype(jnp.float32)
    oh1 = (li == i1).astype(jnp.float32)
    c0 = jnp.sum(oh0, axis=0, keepdims=True)
    counts = c0 + jnp.sum(oh1, axis=0, keepdims=True)
    tiles = jnp.floor((counts + (M - 1)) * (1.0 / M))
    ut = (jax.lax.broadcasted_iota(jnp.int32, (LANES, LANES), 0)
          <= jax.lax.broadcasted_iota(jnp.int32, (LANES, LANES), 1)
          ).astype(jnp.float32)
    tcum = jnp.dot(tiles, ut, preferred_element_type=jnp.float32)
    po = (tcum - tiles) * M
    tril = (jax.lax.broadcasted_iota(jnp.int32, (T, T), 0)
            > jax.lax.broadcasted_iota(jnp.int32, (T, T), 1)
            ).astype(jnp.bfloat16)
    r0 = jnp.dot(tril, oh0.astype(jnp.bfloat16),
                 preferred_element_type=jnp.float32)
    r1 = jnp.dot(tril, oh1.astype(jnp.bfloat16),
                 preferred_element_type=jnp.float32) + c0
    pos0 = jnp.sum(oh0 * (po + r0), axis=1, keepdims=True)
    pos1 = jnp.sum(oh1 * (po + r1), axis=1, keepdims=True)
    pos_ref[...] = jnp.concatenate(
        [2.0 * pos0, 2.0 * pos0 + 1.0, 2.0 * pos1, 2.0 * pos1 + 1.0],
        axis=1).astype(jnp.int32)
    ri = jax.lax.broadcasted_iota(jnp.int32, (32, 1), 0)
    li1 = jax.lax.broadcasted_iota(jnp.int32, (1, LANES), 1)
    tm = ((tcum <= ri.astype(jnp.float32)) & (li1 < E)).astype(jnp.float32)
    te = jnp.minimum(jnp.sum(tm, axis=1, keepdims=True), E - 1)
    na = jnp.sum(jnp.where(li1 == E - 1, tcum, 0.0), axis=1, keepdims=True)
    meta_ref[...] = jnp.where(ri < NT, te, na).astype(jnp.int32)
    pmean = jnp.mean(probs, axis=0, keepdims=True)
    lb = (E / T) * jnp.sum(counts * pmean, axis=1, keepdims=True)
    lb_ref[...] = lb


def _shared_kernel(x_ref, wi0_ref, wi1_ref, wo_ref, scale_ref, o_ref):
    x = x_ref[...].astype(jnp.bfloat16)
    h0 = jnp.dot(x, wi0_ref[...], preferred_element_type=jnp.float32)
    h1 = jnp.dot(x, wi1_ref[...], preferred_element_type=jnp.float32)
    act = (jax.nn.gelu(h0) * h1).astype(jnp.bfloat16)
    s = jnp.dot(act, wo_ref[...], preferred_element_type=jnp.float32)
    o_ref[...] = _rms(s) * scale_ref[...]


def _gmm_kernel(te_ref, xg_ref, wi0_ref, wi1_ref, wo_ref, y_ref):
    i = pl.program_id(0)

    @pl.when(i < te_ref[NT])
    def _():
        x = xg_ref[...]
        h0 = jnp.dot(x, wi0_ref[0], preferred_element_type=jnp.float32)
        h1 = jnp.dot(x, wi1_ref[0], preferred_element_type=jnp.float32)
        act = (jax.nn.gelu(h0) * h1).astype(jnp.bfloat16)
        y_ref[...] = jnp.dot(act, wo_ref[0],
                             preferred_element_type=jnp.float32
                             ).astype(jnp.bfloat16)


def _final_kernel(yg0_ref, yg1_ref, w_ref, shn_ref, p2_ref, o_ref):
    w = w_ref[...]
    r = (w[:, 0:1] * yg0_ref[0].astype(jnp.float32)
         + w[:, 1:2] * yg1_ref[0].astype(jnp.float32))
    o_ref[...] = _rms(r) * p2_ref[...] + shn_ref[...]


def _as_i32(x, rows):
    return jax.lax.bitcast_convert_type(
        x.reshape(rows, HD // 2, 2), jnp.int32)


def _as_bf16(x):
    return jax.lax.bitcast_convert_type(x, jnp.bfloat16)


def _dispatch_sc(rin_bf, pos_flat):
    mesh = plsc.VectorSubcoreMesh(core_axis_name="c", subcore_axis_name="s")
    nsrc = (2 * T) // SCW

    @pl.kernel(out_type=jax.ShapeDtypeStruct((2 * NP, HD // 2), jnp.int32),
               mesh=mesh)
    def k(rin_hbm, pos_hbm, xg_hbm):
        def body(x_vmem, i_vmem):
            pltpu.sync_copy(x_vmem, xg_hbm.at[i_vmem.at[0]])

        pltpu.emit_pipeline(
            body,
            grid=((2 * A) // SCW,),
            in_specs=[
                pl.BlockSpec((SCW, HD // 2), index_map=lambda g: (g % nsrc, 0)),
                pl.BlockSpec((1, SCW), index_map=lambda g: (0, g)),
            ],
            out_specs=[],
            core_axis_name=("c", "s"),
            dimension_semantics=(pltpu.PARALLEL,),
        )(rin_hbm, pos_hbm)

    return _as_bf16(k(_as_i32(rin_bf, 2 * T), pos_flat))


def _combine_sc(y, pos_flat):
    mesh = plsc.VectorSubcoreMesh(core_axis_name="c", subcore_axis_name="s")

    @pl.kernel(out_type=jax.ShapeDtypeStruct((2 * A, HD // 2), jnp.int32),
               mesh=mesh)
    def k(y_hbm, pos_hbm, yg_hbm):
        def body(i_vmem, o_vmem):
            pltpu.sync_copy(y_hbm.at[i_vmem.at[0]], o_vmem)

        pltpu.emit_pipeline(
            body,
            grid=((2 * A) // SCW,),
            in_specs=[pl.BlockSpec((1, SCW), index_map=lambda g: (0, g))],
            out_specs=[pl.BlockSpec((SCW, HD // 2), index_map=lambda g: (g, 0))],
            core_axis_name=("c", "s"),
            dimension_semantics=(pltpu.PARALLEL,),
        )(pos_hbm, yg_hbm)

    return _as_bf16(k(_as_i32(y, 2 * NP), pos_flat))


def kernel(inputs, original_inputs, shared_wi0, shared_wi1, shared_wo,
           post1_scale, pre2_scale, post2_scale, pre_forward_scale,
           router_w, wi0, wi1, wo):
    x = inputs.reshape(T, D)
    orig = original_inputs.reshape(T, D)
    rw = jnp.pad(router_w, ((0, 0), (0, LANES - E)))

    rin_bf, pos, topw, meta, lb = pl.pallas_call(
        _router_kernel,
        out_shape=[
            jax.ShapeDtypeStruct((T, D), jnp.bfloat16),
            jax.ShapeDtypeStruct((T, 4), jnp.int32),
            jax.ShapeDtypeStruct((T, 2), jnp.float32),
            jax.ShapeDtypeStruct((32, 1), jnp.int32),
            jax.ShapeDtypeStruct((1, 1), jnp.float32),
        ],
    )(orig, pre2_scale.reshape(1, D), pre_forward_scale.reshape(1, D), rw)

    pos_flat = jnp.concatenate(
        [pos[:, 0:2].reshape(1, 2 * T), pos[:, 2:4].reshape(1, 2 * T)],
        axis=1)
    xg = _dispatch_sc(rin_bf, pos_flat).reshape(NP, D)

    shn = pl.pallas_call(
        _shared_kernel,
        grid=(T // M,),
        in_specs=[
            pl.BlockSpec((M, D), lambda i: (i, 0)),
            pl.BlockSpec((D, FS), lambda i: (0, 0)),
            pl.BlockSpec((D, FS), lambda i: (0, 0)),
            pl.BlockSpec((FS, D), lambda i: (0, 0)),
            pl.BlockSpec((1, D), lambda i: (0, 0)),
        ],
        out_specs=pl.BlockSpec((M, D), lambda i: (i, 0)),
        out_shape=jax.ShapeDtypeStruct((T, D), jnp.float32),
    )(x, shared_wi0.astype(jnp.bfloat16), shared_wi1.astype(jnp.bfloat16),
      shared_wo.astype(jnp.bfloat16), post1_scale.reshape(1, D))

    meta_flat = meta.reshape(32)[:NT + 1]
    y = pl.pallas_call(
        _gmm_kernel,
        grid_spec=pltpu.PrefetchScalarGridSpec(
            num_scalar_prefetch=1,
            grid=(NT,),
            in_specs=[
                pl.BlockSpec((M, D),
                             lambda i, te: (jnp.minimum(i, te[NT] - 1), 0)),
                pl.BlockSpec((1, D, F),
                             lambda i, te: (te[jnp.minimum(i, te[NT] - 1)],
                                            0, 0)),
                pl.BlockSpec((1, D, F),
                             lambda i, te: (te[jnp.minimum(i, te[NT] - 1)],
                                            0, 0)),
                pl.BlockSpec((1, F, D),
                             lambda i, te: (te[jnp.minimum(i, te[NT] - 1)],
                                            0, 0)),
            ],
            out_specs=pl.BlockSpec((M, D),
                                   lambda i, te: (jnp.minimum(i, te[NT] - 1),
                                                  0)),
        ),
        out_shape=jax.ShapeDtypeStruct((NP, D), jnp.bfloat16),
    )(meta_flat, xg, wi0.astype(jnp.bfloat16), wi1.astype(jnp.bfloat16),
      wo.astype(jnp.bfloat16))

    yg = _combine_sc(y, pos_flat).reshape(2, T, D)

    out = pl.pallas_call(
        _final_kernel,
        grid=(T // M,),
        in_specs=[
            pl.BlockSpec((1, M, D), lambda i: (0, i, 0)),
            pl.BlockSpec((1, M, D), lambda i: (1, i, 0)),
            pl.BlockSpec((M, 2), lambda i: (i, 0)),
            pl.BlockSpec((M, D), lambda i: (i, 0)),
            pl.BlockSpec((1, D), lambda i: (0, 0)),
        ],
        out_specs=pl.BlockSpec((M, D), lambda i: (i, 0)),
        out_shape=jax.ShapeDtypeStruct((T, D), jnp.float32),
    )(yg, yg, topw, shn, post2_scale.reshape(1, D))

    return out.reshape(1, T, D), lb.reshape(())


# speedup vs baseline: 23.9481x; 23.9481x over previous
import jax
import jax.numpy as jnp
from jax.experimental import pallas as pl
from jax.experimental.pallas import tpu as pltpu

T, D, E, K = 2048, 1024, 8, 2
F, FS = 1024, 4096
A = T * K
M = 256
NT = 23
NP = NT * M
EPS = 1e-6
LANES = 128


def _rms(x):
    var = jnp.mean(x * x, axis=-1, keepdims=True)
    return x * jax.lax.rsqrt(var + EPS)


def _router_kernel(orig_ref, pre2_ref, pref_ref, rw_ref,
                   rin_ref, pos_ref, topw_ref, meta_ref, lb_ref):
    x = orig_ref[...]
    xn = _rms(x)
    rin_ref[...] = (xn * pre2_ref[...]).astype(jnp.bfloat16)
    gate = xn * (D ** -0.5) * pref_ref[...]
    logits = jnp.dot(gate, rw_ref[...], preferred_element_type=jnp.float32)
    li = jax.lax.broadcasted_iota(jnp.int32, (T, LANES), 1)
    lmask = li < E
    lm = jnp.where(lmask, logits, -1e30)
    mx = jnp.max(lm, axis=1, keepdims=True)
    ex = jnp.where(lmask, jnp.exp(lm - mx), 0.0)
    probs = ex / jnp.sum(ex, axis=1, keepdims=True)
    p0 = jnp.max(probs, axis=1, keepdims=True)
    i0 = jnp.min(jnp.where(probs == p0, li, LANES), axis=1, keepdims=True)
    pmask = jnp.where(li == i0, -1.0, probs)
    p1 = jnp.max(pmask, axis=1, keepdims=True)
    i1 = jnp.min(jnp.where(pmask == p1, li, LANES), axis=1, keepdims=True)
    s = p0 + p1
    topw_ref[...] = jnp.concatenate([p0 / s, p1 / s], axis=1)
    oh0 = (li == i0).astype(jnp.float32)
    oh1 = (li == i1).astype(jnp.float32)
    c0 = jnp.sum(oh0, axis=0, keepdims=True)
    counts = c0 + jnp.sum(oh1, axis=0, keepdims=True)
    tiles = jnp.floor((counts + (M - 1)) * (1.0 / M))
    ut = (jax.lax.broadcasted_iota(jnp.int32, (LANES, LANES), 0)
          <= jax.lax.broadcasted_iota(jnp.int32, (LANES, LANES), 1)
          ).astype(jnp.float32)
    tcum = jnp.dot(tiles, ut, preferred_element_type=jnp.float32)
    po = (tcum - tiles) * M
    tril = (jax.lax.broadcasted_iota(jnp.int32, (T, T), 0)
            > jax.lax.broadcasted_iota(jnp.int32, (T, T), 1)
            ).astype(jnp.bfloat16)
    r0 = jnp.dot(tril, oh0.astype(jnp.bfloat16),
                 preferred_element_type=jnp.float32)
    r1 = jnp.dot(tril, oh1.astype(jnp.bfloat16),
                 preferred_element_type=jnp.float32) + c0
    pos0 = jnp.sum(oh0 * (po + r0), axis=1, keepdims=True)
    pos1 = jnp.sum(oh1 * (po + r1), axis=1, keepdims=True)
    pos_ref[...] = jnp.concatenate([pos0, pos1], axis=1).astype(jnp.int32)
    ri = jax.lax.broadcasted_iota(jnp.int32, (32, 1), 0)
    li1 = jax.lax.broadcasted_iota(jnp.int32, (1, LANES), 1)
    tm = ((tcum <= ri.astype(jnp.float32)) & (li1 < E)).astype(jnp.float32)
    te = jnp.minimum(jnp.sum(tm, axis=1, keepdims=True), E - 1)
    na = jnp.sum(jnp.where(li1 == E - 1, tcum, 0.0), axis=1, keepdims=True)
    meta_ref[...] = jnp.where(ri < NT, te, na).astype(jnp.int32)
    pmean = jnp.mean(probs, axis=0, keepdims=True)
    lb = (E / T) * jnp.sum(counts * pmean, axis=1, keepdims=True)
    lb_ref[...] = lb


def _shared_kernel(x_ref, wi0_ref, wi1_ref, wo_ref, scale_ref, o_ref):
    x = x_ref[...].astype(jnp.bfloat16)
    h0 = jnp.dot(x, wi0_ref[...], preferred_element_type=jnp.float32)
    h1 = jnp.dot(x, wi1_ref[...], preferred_element_type=jnp.float32)
    act = (jax.nn.gelu(h0) * h1).astype(jnp.bfloat16)
    s = jnp.dot(act, wo_ref[...], preferred_element_type=jnp.float32)
    o_ref[...] = _rms(s) * scale_ref[...]


def _gmm_kernel(te_ref, rin_ref, posr_ref, pos_ref, topw_ref,
                wi0_ref, wi1_ref, wo_ref, routed_ref):
    i = pl.program_id(0)

    @pl.when(i == 0)
    def _():
        routed_ref[...] = jnp.zeros_like(routed_ref)

    @pl.when(i < te_ref[NT])
    def _():
        base = i * M
        pg_col = base + jax.lax.broadcasted_iota(jnp.int32, (M, 1), 0)
        pos0r = posr_ref[0:1, :]
        pos1r = posr_ref[1:2, :]
        sel = ((pos0r == pg_col) | (pos1r == pg_col)).astype(jnp.bfloat16)
        x = jnp.dot(sel, rin_ref[...], preferred_element_type=jnp.float32
                    ).astype(jnp.bfloat16)
        h0 = jnp.dot(x, wi0_ref[0], preferred_element_type=jnp.float32)
        h1 = jnp.dot(x, wi1_ref[0], preferred_element_type=jnp.float32)
        act = (jax.nn.gelu(h0) * h1).astype(jnp.bfloat16)
        y = jnp.dot(act, wo_ref[0], preferred_element_type=jnp.float32
                    ).astype(jnp.bfloat16)
        pg_row = base + jax.lax.broadcasted_iota(jnp.int32, (1, M), 1)
        pos0c = pos_ref[:, 0:1]
        pos1c = pos_ref[:, 1:2]
        cwt = (jnp.where(pos0c == pg_row, topw_ref[:, 0:1], 0.0)
               + jnp.where(pos1c == pg_row, topw_ref[:, 1:2], 0.0)
               ).astype(jnp.bfloat16)
        routed_ref[...] += jnp.dot(cwt, y, preferred_element_type=jnp.float32)


def _final_kernel(routed_ref, shn_ref, p2_ref, o_ref):
    o_ref[...] = _rms(routed_ref[...]) * p2_ref[...] + shn_ref[...]


def kernel(inputs, original_inputs, shared_wi0, shared_wi1, shared_wo,
           post1_scale, pre2_scale, post2_scale, pre_forward_scale,
           router_w, wi0, wi1, wo):
    x = inputs.reshape(T, D)
    orig = original_inputs.reshape(T, D)
    rw = jnp.pad(router_w, ((0, 0), (0, LANES - E)))

    rin_bf, pos, topw, meta, lb = pl.pallas_call(
        _router_kernel,
        out_shape=[
            jax.ShapeDtypeStruct((T, D), jnp.bfloat16),
            jax.ShapeDtypeStruct((T, 2), jnp.int32),
            jax.ShapeDtypeStruct((T, 2), jnp.float32),
            jax.ShapeDtypeStruct((32, 1), jnp.int32),
            jax.ShapeDtypeStruct((1, 1), jnp.float32),
        ],
    )(orig, pre2_scale.reshape(1, D), pre_forward_scale.reshape(1, D), rw)

    shn = pl.pallas_call(
        _shared_kernel,
        grid=(T // M,),
        in_specs=[
            pl.BlockSpec((M, D), lambda i: (i, 0)),
            pl.BlockSpec((D, FS), lambda i: (0, 0)),
            pl.BlockSpec((D, FS), lambda i: (0, 0)),
            pl.BlockSpec((FS, D), lambda i: (0, 0)),
            pl.BlockSpec((1, D), lambda i: (0, 0)),
        ],
        out_specs=pl.BlockSpec((M, D), lambda i: (i, 0)),
        out_shape=jax.ShapeDtypeStruct((T, D), jnp.float32),
    )(x, shared_wi0.astype(jnp.bfloat16), shared_wi1.astype(jnp.bfloat16),
      shared_wo.astype(jnp.bfloat16), post1_scale.reshape(1, D))

    meta_flat = meta.reshape(32)[:NT + 1]
    posr = pos.T
    routed = pl.pallas_call(
        _gmm_kernel,
        grid_spec=pltpu.PrefetchScalarGridSpec(
            num_scalar_prefetch=1,
            grid=(NT,),
            in_specs=[
                pl.BlockSpec((T, D), lambda i, te: (0, 0)),
                pl.BlockSpec((2, T), lambda i, te: (0, 0)),
                pl.BlockSpec((T, 2), lambda i, te: (0, 0)),
                pl.BlockSpec((T, 2), lambda i, te: (0, 0)),
                pl.BlockSpec((1, D, F),
                             lambda i, te: (te[jnp.minimum(i, te[NT] - 1)],
                                            0, 0)),
                pl.BlockSpec((1, D, F),
                             lambda i, te: (te[jnp.minimum(i, te[NT] - 1)],
                                            0, 0)),
                pl.BlockSpec((1, F, D),
                             lambda i, te: (te[jnp.minimum(i, te[NT] - 1)],
                                            0, 0)),
            ],
            out_specs=pl.BlockSpec((T, D), lambda i, te: (0, 0)),
        ),
        out_shape=jax.ShapeDtypeStruct((T, D), jnp.float32),
    )(meta_flat, rin_bf, posr, pos, topw, wi0.astype(jnp.bfloat16),
      wi1.astype(jnp.bfloat16), wo.astype(jnp.bfloat16))

    out = pl.pallas_call(
        _final_kernel,
        grid=(T // M,),
        in_specs=[
            pl.BlockSpec((M, D), lambda i: (i, 0)),
            pl.BlockSpec((M, D), lambda i: (i, 0)),
            pl.BlockSpec((1, D), lambda i: (0, 0)),
        ],
        out_specs=pl.BlockSpec((M, D), lambda i: (i, 0)),
        out_shape=jax.ShapeDtypeStruct((T, D), jnp.float32),
    )(routed, shn, post2_scale.reshape(1, D))

    return out.reshape(1, T, D), lb.reshape(())


# speedup vs baseline: 29.8239x; 1.2454x over previous
import jax
import jax.numpy as jnp
from jax.experimental import pallas as pl
from jax.experimental.pallas import tpu as pltpu

T, D, E, K = 2048, 1024, 8, 2
F, FS = 1024, 4096
A = T * K
M = 256
NT = 23
NP = NT * M
EPS = 1e-6
LANES = 128


def _rms(x):
    var = jnp.mean(x * x, axis=-1, keepdims=True)
    return x * jax.lax.rsqrt(var + EPS)


def _router_kernel(orig_ref, pre2_ref, pref_ref, rw_ref,
                   rin_ref, pos_ref, topw_ref, meta_ref, lb_ref):
    x = orig_ref[...]
    xn = _rms(x)
    rin_ref[...] = (xn * pre2_ref[...]).astype(jnp.bfloat16)
    gate = xn * (D ** -0.5) * pref_ref[...]
    logits = jnp.dot(gate, rw_ref[...], preferred_element_type=jnp.float32)
    li = jax.lax.broadcasted_iota(jnp.int32, (T, LANES), 1)
    lmask = li < E
    lm = jnp.where(lmask, logits, -1e30)
    mx = jnp.max(lm, axis=1, keepdims=True)
    ex = jnp.where(lmask, jnp.exp(lm - mx), 0.0)
    probs = ex / jnp.sum(ex, axis=1, keepdims=True)
    p0 = jnp.max(probs, axis=1, keepdims=True)
    i0 = jnp.min(jnp.where(probs == p0, li, LANES), axis=1, keepdims=True)
    pmask = jnp.where(li == i0, -1.0, probs)
    p1 = jnp.max(pmask, axis=1, keepdims=True)
    i1 = jnp.min(jnp.where(pmask == p1, li, LANES), axis=1, keepdims=True)
    s = p0 + p1
    topw_ref[...] = jnp.concatenate([p0 / s, p1 / s], axis=1)
    oh0 = (li == i0).astype(jnp.float32)
    oh1 = (li == i1).astype(jnp.float32)
    c0 = jnp.sum(oh0, axis=0, keepdims=True)
    counts = c0 + jnp.sum(oh1, axis=0, keepdims=True)
    tiles = jnp.floor((counts + (M - 1)) * (1.0 / M))
    ut = (jax.lax.broadcasted_iota(jnp.int32, (LANES, LANES), 0)
          <= jax.lax.broadcasted_iota(jnp.int32, (LANES, LANES), 1)
          ).astype(jnp.float32)
    tcum = jnp.dot(tiles, ut, preferred_element_type=jnp.float32)
    po = (tcum - tiles) * M
    tril = (jax.lax.broadcasted_iota(jnp.int32, (T, T), 0)
            > jax.lax.broadcasted_iota(jnp.int32, (T, T), 1)
            ).astype(jnp.bfloat16)
    r0 = jnp.dot(tril, oh0.astype(jnp.bfloat16),
                 preferred_element_type=jnp.float32)
    r1 = jnp.dot(tril, oh1.astype(jnp.bfloat16),
                 preferred_element_type=jnp.float32) + c0
    pos0 = jnp.sum(oh0 * (po + r0), axis=1, keepdims=True)
    pos1 = jnp.sum(oh1 * (po + r1), axis=1, keepdims=True)
    pos_ref[...] = jnp.concatenate([pos0, pos1], axis=1).astype(jnp.int32)
    ri = jax.lax.broadcasted_iota(jnp.int32, (32, 1), 0)
    li1 = jax.lax.broadcasted_iota(jnp.int32, (1, LANES), 1)
    tm = ((tcum <= ri.astype(jnp.float32)) & (li1 < E)).astype(jnp.float32)
    te = jnp.minimum(jnp.sum(tm, axis=1, keepdims=True), E - 1)
    na = jnp.sum(jnp.where(li1 == E - 1, tcum, 0.0), axis=1, keepdims=True)
    meta_ref[...] = jnp.where(ri < NT, te, na).astype(jnp.int32)
    pmean = jnp.mean(probs, axis=0, keepdims=True)
    lb = (E / T) * jnp.sum(counts * pmean, axis=1, keepdims=True)
    lb_ref[...] = lb


FB = 512


def _shared_kernel(x_ref, wi0_ref, wi1_ref, wo_ref, scale_ref, o_ref):
    f = pl.program_id(0)
    x = x_ref[...].astype(jnp.bfloat16)
    h0 = jnp.dot(x, wi0_ref[...].astype(jnp.bfloat16),
                 preferred_element_type=jnp.float32)
    h1 = jnp.dot(x, wi1_ref[...].astype(jnp.bfloat16),
                 preferred_element_type=jnp.float32)
    act = (jax.nn.gelu(h0) * h1).astype(jnp.bfloat16)
    contrib = jnp.dot(act, wo_ref[...].astype(jnp.bfloat16),
                      preferred_element_type=jnp.float32)

    @pl.when(f == 0)
    def _():
        o_ref[...] = contrib

    @pl.when(f > 0)
    def _():
        o_ref[...] += contrib

    @pl.when(f == FS // FB - 1)
    def _():
        o_ref[...] = _rms(o_ref[...]) * scale_ref[...]


def _gmm_kernel(te_ref, rin_ref, posr_ref, pos_ref, topw_ref,
                wi0_ref, wi1_ref, wo_ref, routed_ref,
                wi0_bf, wi1_bf, wo_bf):
    i = pl.program_id(0)

    @pl.when(i == 0)
    def _():
        routed_ref[...] = jnp.zeros_like(routed_ref)

    na1 = te_ref[NT] - 1
    cur = te_ref[jnp.minimum(i, na1)]
    prev = te_ref[jnp.minimum(jnp.maximum(i, 1) - 1, na1)]
    @pl.when((i == 0) | (cur != prev))
    def _():
        wi0_bf[...] = wi0_ref[0].astype(jnp.bfloat16)
        wi1_bf[...] = wi1_ref[0].astype(jnp.bfloat16)
        wo_bf[...] = wo_ref[0].astype(jnp.bfloat16)

    @pl.when(i < te_ref[NT])
    def _():
        base = i * M
        pg_col = base + jax.lax.broadcasted_iota(jnp.int32, (M, 1), 0)
        pos0r = posr_ref[0:1, :]
        pos1r = posr_ref[1:2, :]
        sel = ((pos0r == pg_col) | (pos1r == pg_col)).astype(jnp.bfloat16)
        x = jnp.dot(sel, rin_ref[...], preferred_element_type=jnp.float32
                    ).astype(jnp.bfloat16)
        h0 = jnp.dot(x, wi0_bf[...], preferred_element_type=jnp.float32)
        h1 = jnp.dot(x, wi1_bf[...], preferred_element_type=jnp.float32)
        act = (jax.nn.gelu(h0) * h1).astype(jnp.bfloat16)
        y = jnp.dot(act, wo_bf[...], preferred_element_type=jnp.float32
                    ).astype(jnp.bfloat16)
        pg_row = base + jax.lax.broadcasted_iota(jnp.int32, (1, M), 1)
        pos0c = pos_ref[:, 0:1]
        pos1c = pos_ref[:, 1:2]
        cwt = (jnp.where(pos0c == pg_row, topw_ref[:, 0:1], 0.0)
               + jnp.where(pos1c == pg_row, topw_ref[:, 1:2], 0.0)
               ).astype(jnp.bfloat16)
        routed_ref[...] += jnp.dot(cwt, y, preferred_element_type=jnp.float32)


def _final_kernel(routed_ref, shn_ref, p2_ref, o_ref):
    o_ref[...] = _rms(routed_ref[...]) * p2_ref[...] + shn_ref[...]


def kernel(inputs, original_inputs, shared_wi0, shared_wi1, shared_wo,
           post1_scale, pre2_scale, post2_scale, pre_forward_scale,
           router_w, wi0, wi1, wo):
    x = inputs.reshape(T, D)
    orig = original_inputs.reshape(T, D)
    rw = jnp.pad(router_w, ((0, 0), (0, LANES - E)))

    rin_bf, pos, topw, meta, lb = pl.pallas_call(
        _router_kernel,
        out_shape=[
            jax.ShapeDtypeStruct((T, D), jnp.bfloat16),
            jax.ShapeDtypeStruct((T, 2), jnp.int32),
            jax.ShapeDtypeStruct((T, 2), jnp.float32),
            jax.ShapeDtypeStruct((32, 1), jnp.int32),
            jax.ShapeDtypeStruct((1, 1), jnp.float32),
        ],
    )(orig, pre2_scale.reshape(1, D), pre_forward_scale.reshape(1, D), rw)

    shn = pl.pallas_call(
        _shared_kernel,
        grid=(FS // FB,),
        in_specs=[
            pl.BlockSpec((T, D), lambda f: (0, 0)),
            pl.BlockSpec((D, FB), lambda f: (0, f)),
            pl.BlockSpec((D, FB), lambda f: (0, f)),
            pl.BlockSpec((FB, D), lambda f: (f, 0)),
            pl.BlockSpec((1, D), lambda f: (0, 0)),
        ],
        out_specs=pl.BlockSpec((T, D), lambda f: (0, 0)),
        out_shape=jax.ShapeDtypeStruct((T, D), jnp.float32),
    )(x, shared_wi0, shared_wi1, shared_wo, post1_scale.reshape(1, D))

    meta_flat = meta.reshape(32)[:NT + 1]
    posr = pos.T
    routed = pl.pallas_call(
        _gmm_kernel,
        grid_spec=pltpu.PrefetchScalarGridSpec(
            num_scalar_prefetch=1,
            grid=(NT,),
            in_specs=[
                pl.BlockSpec((T, D), lambda i, te: (0, 0)),
                pl.BlockSpec((2, T), lambda i, te: (0, 0)),
                pl.BlockSpec((T, 2), lambda i, te: (0, 0)),
                pl.BlockSpec((T, 2), lambda i, te: (0, 0)),
                pl.BlockSpec((1, D, F),
                             lambda i, te: (te[jnp.minimum(i, te[NT] - 1)],
                                            0, 0)),
                pl.BlockSpec((1, D, F),
                             lambda i, te: (te[jnp.minimum(i, te[NT] - 1)],
                                            0, 0)),
                pl.BlockSpec((1, F, D),
                             lambda i, te: (te[jnp.minimum(i, te[NT] - 1)],
                                            0, 0)),
            ],
            out_specs=pl.BlockSpec((T, D), lambda i, te: (0, 0)),
            scratch_shapes=[
                pltpu.VMEM((D, F), jnp.bfloat16),
                pltpu.VMEM((D, F), jnp.bfloat16),
                pltpu.VMEM((F, D), jnp.bfloat16),
            ],
        ),
        out_shape=jax.ShapeDtypeStruct((T, D), jnp.float32),
    )(meta_flat, rin_bf, posr, pos, topw, wi0, wi1, wo)

    out = pl.pallas_call(
        _final_kernel,
        grid=(T // M,),
        in_specs=[
            pl.BlockSpec((M, D), lambda i: (i, 0)),
            pl.BlockSpec((M, D), lambda i: (i, 0)),
            pl.BlockSpec((1, D), lambda i: (0, 0)),
        ],
        out_specs=pl.BlockSpec((M, D), lambda i: (i, 0)),
        out_shape=jax.ShapeDtypeStruct((T, D), jnp.float32),
    )(routed, shn, post2_scale.reshape(1, D))

    return out.reshape(1, T, D), lb.reshape(())


# speedup vs baseline: 30.5293x; 1.0237x over previous
import jax
import jax.numpy as jnp
from jax.experimental import pallas as pl
from jax.experimental.pallas import tpu as pltpu

T, D, E, K = 2048, 1024, 8, 2
F, FS = 1024, 4096
A = T * K
M = 256
NT = 23
NP = NT * M
EPS = 1e-6
LANES = 128


def _rms(x):
    var = jnp.mean(x * x, axis=-1, keepdims=True)
    return x * jax.lax.rsqrt(var + EPS)


def _router_kernel(orig_ref, pre2_ref, pref_ref, rw_ref,
                   rin_ref, pos_ref, topw_ref, meta_ref, lb_ref):
    x = orig_ref[...]
    xn = _rms(x)
    rin_ref[...] = (xn * pre2_ref[...]).astype(jnp.bfloat16)
    gate = xn * (D ** -0.5) * pref_ref[...]
    logits = jnp.dot(gate, rw_ref[...], preferred_element_type=jnp.float32)
    li = jax.lax.broadcasted_iota(jnp.int32, (T, LANES), 1)
    lmask = li < E
    lm = jnp.where(lmask, logits, -1e30)
    mx = jnp.max(lm, axis=1, keepdims=True)
    ex = jnp.where(lmask, jnp.exp(lm - mx), 0.0)
    probs = ex / jnp.sum(ex, axis=1, keepdims=True)
    p0 = jnp.max(probs, axis=1, keepdims=True)
    i0 = jnp.min(jnp.where(probs == p0, li, LANES), axis=1, keepdims=True)
    pmask = jnp.where(li == i0, -1.0, probs)
    p1 = jnp.max(pmask, axis=1, keepdims=True)
    i1 = jnp.min(jnp.where(pmask == p1, li, LANES), axis=1, keepdims=True)
    s = p0 + p1
    topw_ref[...] = jnp.concatenate([p0 / s, p1 / s], axis=1)
    oh0 = (li == i0).astype(jnp.float32)
    oh1 = (li == i1).astype(jnp.float32)
    c0 = jnp.sum(oh0, axis=0, keepdims=True)
    counts = c0 + jnp.sum(oh1, axis=0, keepdims=True)
    tiles = jnp.floor((counts + (M - 1)) * (1.0 / M))
    ut = (jax.lax.broadcasted_iota(jnp.int32, (LANES, LANES), 0)
          <= jax.lax.broadcasted_iota(jnp.int32, (LANES, LANES), 1)
          ).astype(jnp.float32)
    tcum = jnp.dot(tiles, ut, preferred_element_type=jnp.float32)
    po = (tcum - tiles) * M
    RB = 512
    tril = (jax.lax.broadcasted_iota(jnp.int32, (RB, RB), 0)
            > jax.lax.broadcasted_iota(jnp.int32, (RB, RB), 1)
            ).astype(jnp.bfloat16)
    off = jnp.zeros((1, LANES), jnp.float32)
    ranks = []
    for oh in (oh0, oh1):
        blocks = []
        for b in range(T // RB):
            ohb = oh[b * RB:(b + 1) * RB, :]
            blocks.append(jnp.dot(tril, ohb.astype(jnp.bfloat16),
                                  preferred_element_type=jnp.float32) + off)
            off = off + jnp.sum(ohb, axis=0, keepdims=True)
        ranks.append(jnp.concatenate(blocks, axis=0))
    r0, r1 = ranks
    pos0 = jnp.sum(oh0 * (po + r0), axis=1, keepdims=True)
    pos1 = jnp.sum(oh1 * (po + r1), axis=1, keepdims=True)
    pos_ref[...] = jnp.concatenate([pos0, pos1], axis=1).astype(jnp.int32)
    ri = jax.lax.broadcasted_iota(jnp.int32, (32, 1), 0)
    li1 = jax.lax.broadcasted_iota(jnp.int32, (1, LANES), 1)
    tm = ((tcum <= ri.astype(jnp.float32)) & (li1 < E)).astype(jnp.float32)
    te = jnp.minimum(jnp.sum(tm, axis=1, keepdims=True), E - 1)
    na = jnp.sum(jnp.where(li1 == E - 1, tcum, 0.0), axis=1, keepdims=True)
    meta_ref[...] = jnp.where(ri < NT, te, na).astype(jnp.int32)
    pmean = jnp.mean(probs, axis=0, keepdims=True)
    lb = (E / T) * jnp.sum(counts * pmean, axis=1, keepdims=True)
    lb_ref[...] = lb


FB = 512


def _shared_kernel(x_ref, wi0_ref, wi1_ref, wo_ref, scale_ref, o_ref):
    f = pl.program_id(0)
    x = x_ref[...].astype(jnp.bfloat16)
    h0 = jnp.dot(x, wi0_ref[...].astype(jnp.bfloat16),
                 preferred_element_type=jnp.float32)
    h1 = jnp.dot(x, wi1_ref[...].astype(jnp.bfloat16),
                 preferred_element_type=jnp.float32)
    act = (jax.nn.gelu(h0) * h1).astype(jnp.bfloat16)
    contrib = jnp.dot(act, wo_ref[...].astype(jnp.bfloat16),
                      preferred_element_type=jnp.float32)

    @pl.when(f == 0)
    def _():
        o_ref[...] = contrib

    @pl.when(f > 0)
    def _():
        o_ref[...] += contrib

    @pl.when(f == FS // FB - 1)
    def _():
        o_ref[...] = _rms(o_ref[...]) * scale_ref[...]


def _gmm_kernel(te_ref, rin_ref, posr_ref, pos_ref, topw_ref,
                wi0_ref, wi1_ref, wo_ref, routed_ref,
                wi0_bf, wi1_bf, wo_bf):
    i = pl.program_id(0)

    @pl.when(i == 0)
    def _():
        routed_ref[...] = jnp.zeros_like(routed_ref)

    na1 = te_ref[NT] - 1
    cur = te_ref[jnp.minimum(i, na1)]
    prev = te_ref[jnp.minimum(jnp.maximum(i, 1) - 1, na1)]
    @pl.when((i == 0) | (cur != prev))
    def _():
        wi0_bf[...] = wi0_ref[0].astype(jnp.bfloat16)
        wi1_bf[...] = wi1_ref[0].astype(jnp.bfloat16)
        wo_bf[...] = wo_ref[0].astype(jnp.bfloat16)

    @pl.when(i < te_ref[NT])
    def _():
        base = i * M
        pg_col = base + jax.lax.broadcasted_iota(jnp.int32, (M, 1), 0)
        pos0r = posr_ref[0:1, :]
        pos1r = posr_ref[1:2, :]
        sel = ((pos0r == pg_col) | (pos1r == pg_col)).astype(jnp.bfloat16)
        x = jnp.dot(sel, rin_ref[...], preferred_element_type=jnp.float32
                    ).astype(jnp.bfloat16)
        h0 = jnp.dot(x, wi0_bf[...], preferred_element_type=jnp.float32)
        h1 = jnp.dot(x, wi1_bf[...], preferred_element_type=jnp.float32)
        act = (jax.nn.gelu(h0) * h1).astype(jnp.bfloat16)
        y = jnp.dot(act, wo_bf[...], preferred_element_type=jnp.float32
                    ).astype(jnp.bfloat16)
        pg_row = base + jax.lax.broadcasted_iota(jnp.int32, (1, M), 1)
        pos0c = pos_ref[:, 0:1]
        pos1c = pos_ref[:, 1:2]
        cwt = (jnp.where(pos0c == pg_row, topw_ref[:, 0:1], 0.0)
               + jnp.where(pos1c == pg_row, topw_ref[:, 1:2], 0.0)
               ).astype(jnp.bfloat16)
        routed_ref[...] += jnp.dot(cwt, y, preferred_element_type=jnp.float32)


def _final_kernel(routed_ref, shn_ref, p2_ref, o_ref):
    o_ref[...] = _rms(routed_ref[...]) * p2_ref[...] + shn_ref[...]


def kernel(inputs, original_inputs, shared_wi0, shared_wi1, shared_wo,
           post1_scale, pre2_scale, post2_scale, pre_forward_scale,
           router_w, wi0, wi1, wo):
    x = inputs.reshape(T, D)
    orig = original_inputs.reshape(T, D)
    rw = jnp.pad(router_w, ((0, 0), (0, LANES - E)))

    rin_bf, pos, topw, meta, lb = pl.pallas_call(
        _router_kernel,
        out_shape=[
            jax.ShapeDtypeStruct((T, D), jnp.bfloat16),
            jax.ShapeDtypeStruct((T, 2), jnp.int32),
            jax.ShapeDtypeStruct((T, 2), jnp.float32),
            jax.ShapeDtypeStruct((32, 1), jnp.int32),
            jax.ShapeDtypeStruct((1, 1), jnp.float32),
        ],
    )(orig, pre2_scale.reshape(1, D), pre_forward_scale.reshape(1, D), rw)

    shn = pl.pallas_call(
        _shared_kernel,
        grid=(FS // FB,),
        in_specs=[
            pl.BlockSpec((T, D), lambda f: (0, 0)),
            pl.BlockSpec((D, FB), lambda f: (0, f)),
            pl.BlockSpec((D, FB), lambda f: (0, f)),
            pl.BlockSpec((FB, D), lambda f: (f, 0)),
            pl.BlockSpec((1, D), lambda f: (0, 0)),
        ],
        out_specs=pl.BlockSpec((T, D), lambda f: (0, 0)),
        out_shape=jax.ShapeDtypeStruct((T, D), jnp.float32),
    )(x, shared_wi0, shared_wi1, shared_wo, post1_scale.reshape(1, D))

    meta_flat = meta.reshape(32)[:NT + 1]
    posr = pos.T
    routed = pl.pallas_call(
        _gmm_kernel,
        grid_spec=pltpu.PrefetchScalarGridSpec(
            num_scalar_prefetch=1,
            grid=(NT,),
            in_specs=[
                pl.BlockSpec((T, D), lambda i, te: (0, 0)),
                pl.BlockSpec((2, T), lambda i, te: (0, 0)),
                pl.BlockSpec((T, 2), lambda i, te: (0, 0)),
                pl.BlockSpec((T, 2), lambda i, te: (0, 0)),
                pl.BlockSpec((1, D, F),
                             lambda i, te: (te[jnp.minimum(i, te[NT] - 1)],
                                            0, 0)),
                pl.BlockSpec((1, D, F),
                             lambda i, te: (te[jnp.minimum(i, te[NT] - 1)],
                                            0, 0)),
                pl.BlockSpec((1, F, D),
                             lambda i, te: (te[jnp.minimum(i, te[NT] - 1)],
                                            0, 0)),
            ],
            out_specs=pl.BlockSpec((T, D), lambda i, te: (0, 0)),
            scratch_shapes=[
                pltpu.VMEM((D, F), jnp.bfloat16),
                pltpu.VMEM((D, F), jnp.bfloat16),
                pltpu.VMEM((F, D), jnp.bfloat16),
            ],
        ),
        out_shape=jax.ShapeDtypeStruct((T, D), jnp.float32),
    )(meta_flat, rin_bf, posr, pos, topw, wi0, wi1, wo)

    out = pl.pallas_call(
        _final_kernel,
        grid=(T // M,),
        in_specs=[
            pl.BlockSpec((M, D), lambda i: (i, 0)),
            pl.BlockSpec((M, D), lambda i: (i, 0)),
            pl.BlockSpec((1, D), lambda i: (0, 0)),
        ],
        out_specs=pl.BlockSpec((M, D), lambda i: (i, 0)),
        out_shape=jax.ShapeDtypeStruct((T, D), jnp.float32),
    )(routed, shn, post2_scale.reshape(1, D))

    return out.reshape(1, T, D), lb.reshape(())


# speedup vs baseline: 31.6247x; 1.0359x over previous
import jax
import jax.numpy as jnp
from jax.experimental import pallas as pl
from jax.experimental.pallas import tpu as pltpu

T, D, E, K = 2048, 1024, 8, 2
F, FS = 1024, 4096
A = T * K
M = 256
NT = 23
NP = NT * M
EPS = 1e-6
LANES = 128


def _rms(x):
    var = jnp.mean(x * x, axis=-1, keepdims=True)
    return x * jax.lax.rsqrt(var + EPS)


def _router_kernel(orig_ref, pre2_ref, pref_ref, rw_ref,
                   rin_ref, pos_ref, topw_ref, meta_ref, lb_ref):
    x = orig_ref[...]
    xn = _rms(x)
    rin_ref[...] = (xn * pre2_ref[...]).astype(jnp.bfloat16)
    gate = xn * (D ** -0.5) * pref_ref[...]
    logits = jnp.dot(gate, rw_ref[...], preferred_element_type=jnp.float32)
    li = jax.lax.broadcasted_iota(jnp.int32, (T, LANES), 1)
    lmask = li < E
    lm = jnp.where(lmask, logits, -1e30)
    mx = jnp.max(lm, axis=1, keepdims=True)
    ex = jnp.where(lmask, jnp.exp(lm - mx), 0.0)
    probs = ex / jnp.sum(ex, axis=1, keepdims=True)
    p0 = jnp.max(probs, axis=1, keepdims=True)
    i0 = jnp.min(jnp.where(probs == p0, li, LANES), axis=1, keepdims=True)
    pmask = jnp.where(li == i0, -1.0, probs)
    p1 = jnp.max(pmask, axis=1, keepdims=True)
    i1 = jnp.min(jnp.where(pmask == p1, li, LANES), axis=1, keepdims=True)
    s = p0 + p1
    topw_ref[...] = jnp.concatenate([p0 / s, p1 / s], axis=1)
    oh0 = (li == i0).astype(jnp.float32)
    oh1 = (li == i1).astype(jnp.float32)
    c0 = jnp.sum(oh0, axis=0, keepdims=True)
    counts = c0 + jnp.sum(oh1, axis=0, keepdims=True)
    tiles = jnp.floor((counts + (M - 1)) * (1.0 / M))
    ut = (jax.lax.broadcasted_iota(jnp.int32, (LANES, LANES), 0)
          <= jax.lax.broadcasted_iota(jnp.int32, (LANES, LANES), 1)
          ).astype(jnp.float32)
    tcum = jnp.dot(tiles, ut, preferred_element_type=jnp.float32)
    po = (tcum - tiles) * M
    RB = 512
    tril = (jax.lax.broadcasted_iota(jnp.int32, (RB, RB), 0)
            > jax.lax.broadcasted_iota(jnp.int32, (RB, RB), 1)
            ).astype(jnp.bfloat16)
    off = jnp.zeros((1, LANES), jnp.float32)
    ranks = []
    for oh in (oh0, oh1):
        blocks = []
        for b in range(T // RB):
            ohb = oh[b * RB:(b + 1) * RB, :]
            blocks.append(jnp.dot(tril, ohb.astype(jnp.bfloat16),
                                  preferred_element_type=jnp.float32) + off)
            off = off + jnp.sum(ohb, axis=0, keepdims=True)
        ranks.append(jnp.concatenate(blocks, axis=0))
    r0, r1 = ranks
    pos0 = jnp.sum(oh0 * (po + r0), axis=1, keepdims=True)
    pos1 = jnp.sum(oh1 * (po + r1), axis=1, keepdims=True)
    pos_ref[...] = jnp.concatenate([pos0, pos1], axis=1).astype(jnp.int32)
    ri = jax.lax.broadcasted_iota(jnp.int32, (32, 1), 0)
    li1 = jax.lax.broadcasted_iota(jnp.int32, (1, LANES), 1)
    tm = ((tcum <= ri.astype(jnp.float32)) & (li1 < E)).astype(jnp.float32)
    te = jnp.minimum(jnp.sum(tm, axis=1, keepdims=True), E - 1)
    na = jnp.sum(jnp.where(li1 == E - 1, tcum, 0.0), axis=1, keepdims=True)
    meta_ref[...] = jnp.where(ri < NT, te, na).astype(jnp.int32)
    pmean = jnp.mean(probs, axis=0, keepdims=True)
    lb = (E / T) * jnp.sum(counts * pmean, axis=1, keepdims=True)
    lb_ref[...] = lb


FB = 512


def _shared_kernel(x_ref, wi0_ref, wi1_ref, wo_ref, scale_ref, o_ref,
                   acc_ref):
    f = pl.program_id(0)
    x = x_ref[...].astype(jnp.bfloat16)
    h0 = jnp.dot(x, wi0_ref[...].astype(jnp.bfloat16),
                 preferred_element_type=jnp.float32)
    h1 = jnp.dot(x, wi1_ref[...].astype(jnp.bfloat16),
                 preferred_element_type=jnp.float32)
    act = (jax.nn.gelu(h0) * h1).astype(jnp.bfloat16)
    contrib = jnp.dot(act, wo_ref[...].astype(jnp.bfloat16),
                      preferred_element_type=jnp.float32)

    @pl.when(f == 0)
    def _():
        acc_ref[...] = contrib

    @pl.when(f > 0)
    def _():
        acc_ref[...] += contrib

    @pl.when(f == FS // FB - 1)
    def _():
        o_ref[...] = (_rms(acc_ref[...]) * scale_ref[...]
                      ).astype(jnp.bfloat16)


def _gmm_kernel(te_ref, rin_ref, posr_ref, pos_ref, topw_ref,
                wi0_ref, wi1_ref, wo_ref, shn_ref, p2_ref, o_ref,
                acc_ref, wi0_bf, wi1_bf, wo_bf):
    i = pl.program_id(0)

    @pl.when(i == 0)
    def _():
        acc_ref[...] = jnp.zeros_like(acc_ref)

    na1 = te_ref[NT] - 1
    cur = te_ref[jnp.minimum(i, na1)]
    prev = te_ref[jnp.minimum(jnp.maximum(i, 1) - 1, na1)]
    @pl.when((i == 0) | ((i < NT) & (cur != prev)))
    def _():
        wi0_bf[...] = wi0_ref[0].astype(jnp.bfloat16)
        wi1_bf[...] = wi1_ref[0].astype(jnp.bfloat16)
        wo_bf[...] = wo_ref[0].astype(jnp.bfloat16)

    @pl.when(i < te_ref[NT])
    def _():
        base = i * M
        pg_col = base + jax.lax.broadcasted_iota(jnp.int32, (M, 1), 0)
        pos0r = posr_ref[0:1, :]
        pos1r = posr_ref[1:2, :]
        sel = ((pos0r == pg_col) | (pos1r == pg_col)).astype(jnp.bfloat16)
        x = jnp.dot(sel, rin_ref[...], preferred_element_type=jnp.float32
                    ).astype(jnp.bfloat16)
        h0 = jnp.dot(x, wi0_bf[...], preferred_element_type=jnp.float32)
        h1 = jnp.dot(x, wi1_bf[...], preferred_element_type=jnp.float32)
        act = (jax.nn.gelu(h0) * h1).astype(jnp.bfloat16)
        y = jnp.dot(act, wo_bf[...], preferred_element_type=jnp.float32
                    ).astype(jnp.bfloat16)
        pg_row = base + jax.lax.broadcasted_iota(jnp.int32, (1, M), 1)
        pos0c = pos_ref[:, 0:1]
        pos1c = pos_ref[:, 1:2]
        cwt = (jnp.where(pos0c == pg_row, topw_ref[:, 0:1], 0.0)
               + jnp.where(pos1c == pg_row, topw_ref[:, 1:2], 0.0)
               ).astype(jnp.bfloat16)
        acc_ref[...] += jnp.dot(cwt, y, preferred_element_type=jnp.float32)

    @pl.when(i >= NT)
    def _():
        r = acc_ref[pl.ds((i - NT) * M, M), :]
        o_ref[...] = (_rms(r) * p2_ref[...]
                      + shn_ref[...].astype(jnp.float32))


def kernel(inputs, original_inputs, shared_wi0, shared_wi1, shared_wo,
           post1_scale, pre2_scale, post2_scale, pre_forward_scale,
           router_w, wi0, wi1, wo):
    x = inputs.reshape(T, D)
    orig = original_inputs.reshape(T, D)
    rw = jnp.pad(router_w, ((0, 0), (0, LANES - E)))

    rin_bf, pos, topw, meta, lb = pl.pallas_call(
        _router_kernel,
        out_shape=[
            jax.ShapeDtypeStruct((T, D), jnp.bfloat16),
            jax.ShapeDtypeStruct((T, 2), jnp.int32),
            jax.ShapeDtypeStruct((T, 2), jnp.float32),
            jax.ShapeDtypeStruct((32, 1), jnp.int32),
            jax.ShapeDtypeStruct((1, 1), jnp.float32),
        ],
    )(orig, pre2_scale.reshape(1, D), pre_forward_scale.reshape(1, D), rw)

    shn = pl.pallas_call(
        _shared_kernel,
        grid=(FS // FB,),
        in_specs=[
            pl.BlockSpec((T, D), lambda f: (0, 0)),
            pl.BlockSpec((D, FB), lambda f: (0, f)),
            pl.BlockSpec((D, FB), lambda f: (0, f)),
            pl.BlockSpec((FB, D), lambda f: (f, 0)),
            pl.BlockSpec((1, D), lambda f: (0, 0)),
        ],
        out_specs=pl.BlockSpec((T, D), lambda f: (0, 0)),
        out_shape=jax.ShapeDtypeStruct((T, D), jnp.bfloat16),
        scratch_shapes=[pltpu.VMEM((T, D), jnp.float32)],
    )(x, shared_wi0, shared_wi1, shared_wo, post1_scale.reshape(1, D))

    meta_flat = meta.reshape(32)[:NT + 1]
    posr = pos.T

    def _wmap(i, te):
        return (te[jnp.minimum(i, te[NT] - 1)], 0, 0)

    def _omap(i, te):
        return (jnp.where(i < NT, 0, i - NT), 0)

    out = pl.pallas_call(
        _gmm_kernel,
        grid_spec=pltpu.PrefetchScalarGridSpec(
            num_scalar_prefetch=1,
            grid=(NT + T // M,),
            in_specs=[
                pl.BlockSpec((T, D), lambda i, te: (0, 0)),
                pl.BlockSpec((2, T), lambda i, te: (0, 0)),
                pl.BlockSpec((T, 2), lambda i, te: (0, 0)),
                pl.BlockSpec((T, 2), lambda i, te: (0, 0)),
                pl.BlockSpec((1, D, F), _wmap),
                pl.BlockSpec((1, D, F), _wmap),
                pl.BlockSpec((1, F, D), _wmap),
                pl.BlockSpec((M, D), _omap),
                pl.BlockSpec((1, D), lambda i, te: (0, 0)),
            ],
            out_specs=pl.BlockSpec((M, D), _omap),
            scratch_shapes=[
                pltpu.VMEM((T, D), jnp.float32),
                pltpu.VMEM((D, F), jnp.bfloat16),
                pltpu.VMEM((D, F), jnp.bfloat16),
                pltpu.VMEM((F, D), jnp.bfloat16),
            ],
        ),
        out_shape=jax.ShapeDtypeStruct((T, D), jnp.float32),
    )(meta_flat, rin_bf, posr, pos, topw, wi0, wi1, wo, shn,
      post2_scale.reshape(1, D))

    return out.reshape(1, T, D), lb.reshape(())
